# Initial kernel scaffold; baseline (speedup 1.0000x reference)
import functools

import jax
import jax.numpy as jnp
from jax import lax
from jax.experimental import pallas as pl
from jax.experimental.pallas import tpu as pltpu
from jax.experimental.pallas import tpu_sc as plsc

N_CAT = 26
VOCAB = 100000
D = 32
B = 16384
L = 16

NC, NS = 2, 16
NW = NC * NS
ROWS = B * N_CAT
RPW = ROWS // NW
GCH = 4 * N_CAT
NCH = RPW // GCH
PAT = 8 * N_CAT
PVR = PAT // L


def _body(x_hbm, tab_hbm, offs_hbm, btile_hbm, out_hbm,
          idx_v, rows_v, btile_v, offs_v, sem):
    wid = lax.axis_index("s") * NC + lax.axis_index("c")
    r0 = wid * RPW

    pltpu.sync_copy(x_hbm.at[pl.ds(r0, RPW)], idx_v)
    pltpu.sync_copy(offs_hbm, offs_v)
    pltpu.sync_copy(btile_hbm, btile_v)

    def add_offsets(i, _):
        base = i * PAT
        for k in range(PVR):
            sl = pl.ds(base + k * L, L)
            idx_v[sl] = idx_v[sl] + offs_v[pl.ds(k * L, L)]
        return 0
    lax.fori_loop(0, RPW // PAT, add_offsets, 0)

    def chunk(c, _):
        pltpu.async_copy(tab_hbm.at[idx_v.at[pl.ds(c * GCH, GCH)]],
                         rows_v.at[0], sem).wait()

        def bias_row(r, _):
            for h in range(D // L):
                sl = pl.ds(h * L, L)
                rows_v[0, r, sl] = rows_v[0, r, sl] + btile_v[r, sl]
            return 0
        lax.fori_loop(0, GCH, bias_row, 0)

        pltpu.sync_copy(rows_v.at[0], out_hbm.at[pl.ds(r0 + c * GCH, GCH)])
        return 0
    lax.fori_loop(0, NCH, chunk, 0)


def kernel(x_categ, tables, biases):
    x_flat = x_categ.reshape(-1)
    tab_flat = tables.reshape(N_CAT * VOCAB, D)
    offs = jnp.tile(jnp.arange(N_CAT, dtype=jnp.int32) * VOCAB, PAT // N_CAT)
    btile = jnp.tile(biases, (GCH // N_CAT, 1))

    mesh = plsc.VectorSubcoreMesh(core_axis_name="c", subcore_axis_name="s")
    out = pl.kernel(
        _body,
        mesh=mesh,
        out_type=jax.ShapeDtypeStruct((ROWS, D), jnp.float32),
        scratch_types=[
            pltpu.VMEM((RPW,), jnp.int32),
            pltpu.VMEM((2, GCH, D), jnp.float32),
            pltpu.VMEM((GCH, D), jnp.float32),
            pltpu.VMEM((PAT,), jnp.int32),
            pltpu.SemaphoreType.DMA,
        ],
    )(x_flat, tab_flat, offs, btile)
    return out.reshape(B, N_CAT, D)


# speedup vs baseline: 1.0657x; 1.0657x over previous
import functools

import jax
import jax.numpy as jnp
from jax import lax
from jax.experimental import pallas as pl
from jax.experimental.pallas import tpu as pltpu
from jax.experimental.pallas import tpu_sc as plsc

N_CAT = 26
VOCAB = 100000
D = 32
B = 16384
L = 16

NC, NS = 2, 16
NW = NC * NS
ROWS = B * N_CAT
RPW = ROWS // NW
GCH = 4 * N_CAT
NCH = RPW // GCH
PAT = 8 * N_CAT
PVR = PAT // L


def _body(x_hbm, tab_hbm, offs_hbm, btile_hbm, out_hbm,
          idx_v, rows_v, btile_v, offs_v, sem):
    wid = lax.axis_index("s") * NC + lax.axis_index("c")
    r0 = wid * RPW

    pltpu.sync_copy(x_hbm.at[pl.ds(r0, RPW)], idx_v)
    pltpu.sync_copy(offs_hbm, offs_v)
    pltpu.sync_copy(btile_hbm, btile_v)

    def add_offsets(i, _):
        base = i * PAT
        for k in range(PVR):
            sl = pl.ds(base + k * L, L)
            idx_v[sl] = idx_v[sl] + offs_v[pl.ds(k * L, L)]
        return 0
    lax.fori_loop(0, RPW // PAT, add_offsets, 0)

    def chunk(c, _):
        pltpu.async_copy(tab_hbm.at[idx_v.at[pl.ds(c * GCH, GCH)]],
                         rows_v.at[0], sem).wait()

        def bias_row(r, _):
            for h in range(D // L):
                sl = pl.ds(h * L, L)
                rows_v[0, r, sl] = rows_v[0, r, sl] + btile_v[r, sl]
            return 0
        lax.fori_loop(0, GCH, bias_row, 0)

        pltpu.sync_copy(rows_v.at[0], out_hbm.at[pl.ds(r0 + c * GCH, GCH)])
        return 0
    lax.fori_loop(0, NCH, chunk, 0)


def kernel(x_categ, tables, biases):
    x_flat = x_categ.reshape(-1)
    tab_flat = tables.reshape(N_CAT * VOCAB, D)
    offs = jnp.tile(jnp.arange(N_CAT, dtype=jnp.int32) * VOCAB, PAT // N_CAT)
    btile = jnp.tile(biases, (GCH // N_CAT, 1))

    mesh = plsc.VectorSubcoreMesh(core_axis_name="c", subcore_axis_name="s")
    out = pl.kernel(
        _body,
        mesh=mesh,
        out_type=jax.ShapeDtypeStruct((ROWS, D), jnp.float32),
        compiler_params=pltpu.CompilerParams(use_tc_tiling_on_sc=False),
        scratch_types=[
            pltpu.VMEM((RPW,), jnp.int32),
            pltpu.VMEM((2, GCH, D), jnp.float32),
            pltpu.VMEM((GCH, D), jnp.float32),
            pltpu.VMEM((PAT,), jnp.int32),
            pltpu.SemaphoreType.DMA,
        ],
    )(x_flat, tab_flat, offs, btile)
    return out.reshape(B, N_CAT, D)


# speedup vs baseline: 1.1669x; 1.0950x over previous
import functools

import jax
import jax.numpy as jnp
from jax import lax
from jax.experimental import pallas as pl
from jax.experimental.pallas import tpu as pltpu
from jax.experimental.pallas import tpu_sc as plsc

N_CAT = 26
VOCAB = 100000
D = 32
B = 16384
L = 16

NC, NS = 2, 16
NW = NC * NS
ROWS = B * N_CAT
RPW = ROWS // NW
GCH = 4 * N_CAT
NCH = RPW // GCH
PAT = 8 * N_CAT
PVR = PAT // L


G = 4
NB = NCH // G
UNROLL = 13


def _body(x_hbm, tab_hbm, offs_hbm, btile_hbm, out_hbm,
          idx_v, rows_v, wbuf_v, btile_v, offs_v, sem_g, sem_w):
    wid = lax.axis_index("s") * NC + lax.axis_index("c")
    r0 = wid * RPW

    pltpu.sync_copy(x_hbm.at[pl.ds(r0, RPW)], idx_v)
    pltpu.sync_copy(offs_hbm, offs_v)
    pltpu.sync_copy(btile_hbm, btile_v)

    def add_offsets(i, _):
        base = i * PAT
        for k in range(PVR):
            sl = pl.ds(base + k * L, L)
            idx_v[sl] = idx_v[sl] + offs_v[pl.ds(k * L, L)]
        return 0
    lax.fori_loop(0, RPW // PAT, add_offsets, 0)

    def gather(c, s):
        return pltpu.make_async_copy(
            tab_hbm.at[idx_v.at[pl.ds(c * GCH, GCH)]], rows_v.at[s],
            sem_g.at[s])

    def write(c, s):
        return pltpu.make_async_copy(
            wbuf_v.at[s], out_hbm.at[pl.ds(r0 + c * GCH, GCH)], sem_w.at[s])

    for s in range(G):
        gather(s, s).start()

    def block(c2, _):
        for s in range(G):
            c = c2 * G + s
            gather(c, s).wait()

            @pl.when(c2 > 0)
            def _():
                write(c - G, s).wait()

            def bias_blk(r8, _):
                for r in range(UNROLL):
                    rr = r8 * UNROLL + r
                    for h in range(D // L):
                        sl = pl.ds(h * L, L)
                        wbuf_v[s, rr, sl] = rows_v[s, rr, sl] + btile_v[rr, sl]
                return 0
            lax.fori_loop(0, GCH // UNROLL, bias_blk, 0)

            write(c, s).start()

            @pl.when(c2 < NB - 1)
            def _():
                gather(c + G, s).start()
        return 0
    lax.fori_loop(0, NB, block, 0)

    for s in range(G):
        write((NB - 1) * G + s, s).wait()


def kernel(x_categ, tables, biases):
    x_flat = x_categ.reshape(-1)
    tab_flat = tables.reshape(N_CAT * VOCAB, D)
    offs = jnp.tile(jnp.arange(N_CAT, dtype=jnp.int32) * VOCAB, PAT // N_CAT)
    btile = jnp.tile(biases, (GCH // N_CAT, 1))

    mesh = plsc.VectorSubcoreMesh(core_axis_name="c", subcore_axis_name="s")
    out = pl.kernel(
        _body,
        mesh=mesh,
        out_type=jax.ShapeDtypeStruct((ROWS, D), jnp.float32),
        compiler_params=pltpu.CompilerParams(use_tc_tiling_on_sc=False),
        scratch_types=[
            pltpu.VMEM((RPW,), jnp.int32),
            pltpu.VMEM((G, GCH, D), jnp.float32),
            pltpu.VMEM((G, GCH, D), jnp.float32),
            pltpu.VMEM((GCH, D), jnp.float32),
            pltpu.VMEM((PAT,), jnp.int32),
            pltpu.SemaphoreType.DMA((G,)),
            pltpu.SemaphoreType.DMA((G,)),
        ],
    )(x_flat, tab_flat, offs, btile)
    return out.reshape(B, N_CAT, D)
